# baseline (speedup 1.0000x reference)
import functools

import jax
import jax.numpy as jnp
import numpy as np
from jax import lax
from jax.experimental import pallas as pl
from jax.experimental.pallas import tpu as pltpu
from jax.experimental.pallas import tpu_sc as plsc

N = 10000
DEG = 32
F = 128
K = 4
H = 32
OUT = 128
FW = F // 2

NC = 2
NS = 16
NW = NC * NS
NODES_PER_W = 320
NP = NW * NODES_PER_W
GN = 4
ROWS_PER_CHUNK = GN * DEG
CHUNKS = NODES_PER_W // GN

_PERM = np.array(
    [32 * (p // 32) + (2 * (p % 32) if p % 32 < 16 else 2 * (p % 32 - 16) + 1)
     for p in range(F)], dtype=np.int32)


def _sc_segment_sum(xp, idx_blocks):
    mesh = plsc.VectorSubcoreMesh(
        core_axis_name="c", subcore_axis_name="s", num_cores=NC, num_subcores=NS
    )

    @functools.partial(
        pl.kernel,
        out_type=jax.ShapeDtypeStruct((NP, F), jnp.float32),
        mesh=mesh,
        scratch_types=[
            pltpu.VMEM_SHARED((N, F), jnp.float32),
            pltpu.VMEM((NODES_PER_W * DEG,), jnp.int32),
            pltpu.VMEM((ROWS_PER_CHUNK, F), jnp.float32),
            pltpu.VMEM((ROWS_PER_CHUNK, F), jnp.float32),
            pltpu.VMEM((8, F), jnp.float32),
            pltpu.VMEM((8, F), jnp.float32),
            pltpu.SemaphoreType.DMA,
            pltpu.SemaphoreType.DMA,
            pltpu.SemaphoreType.DMA,
            pltpu.SemaphoreType.DMA,
        ],
    )
    def k(x_hbm, idx_hbm, out_hbm, x_s, idx_v, buf0, buf1, ob0, ob1,
          sem0, sem1, osem0, osem1):
        s = lax.axis_index("s")
        w = s * NC + lax.axis_index("c")
        rows = 624
        pltpu.sync_copy(
            x_hbm.at[pl.ds(s * rows, rows)], x_s.at[pl.ds(s * rows, rows)]
        )

        @pl.when(s == 0)
        def _():
            pltpu.sync_copy(
                x_hbm.at[pl.ds(NS * rows, N - NS * rows)],
                x_s.at[pl.ds(NS * rows, N - NS * rows)],
            )
        IPW = NODES_PER_W * DEG

        @pl.when(w < NW - 1)
        def _():
            pltpu.sync_copy(idx_hbm.at[pl.ds(w * IPW, IPW)], idx_v)

        @pl.when(w == NW - 1)
        def _():
            real = N * DEG - (NW - 1) * IPW
            pltpu.sync_copy(
                idx_hbm.at[pl.ds((NW - 1) * IPW, real)], idx_v.at[pl.ds(0, real)]
            )
            zeros16 = jnp.zeros((16,), jnp.int32)

            def zfill(i, _):
                idx_v[pl.ds(real + i * 16, 16)] = zeros16
                return 0
            lax.fori_loop(0, (IPW - real) // 16, zfill, 0)
        plsc.subcore_barrier()
        bufs = (buf0, buf1)
        sems = (sem0, sem1)
        obs = (ob0, ob1)
        osems = (osem0, osem1)

        pltpu.async_copy(x_s.at[idx_v.at[pl.ds(0, ROWS_PER_CHUNK)]], buf0, sem0)
        pltpu.async_copy(x_s.at[idx_v.at[pl.ds(ROWS_PER_CHUNK, ROWS_PER_CHUNK)]], buf1, sem1)

        def reduce_chunk(buf, ob, node0):
            for g in range(GN):
                def red(d, accs):
                    return tuple(
                        accs[cb] + buf[g * DEG + d, pl.ds(cb * 16, 16)]
                        for cb in range(8)
                    )
                accs = lax.fori_loop(
                    0, DEG, red,
                    tuple(jnp.zeros((16,), jnp.float32) for _ in range(8)),
                )
                for cb in range(8):
                    ob[node0 + g, pl.ds(cb * 16, 16)] = accs[cb]

        def outer(t2, _):
            for p in range(2):
                base = t2 * 16 + p * 8

                @pl.when(t2 > 0)
                def _():
                    pltpu.make_async_copy(
                        obs[p], out_hbm.at[pl.ds(0, 8)], osems[p]
                    ).wait()
                for b in range(2):
                    j = t2 * 4 + p * 2 + b
                    pltpu.make_async_copy(
                        x_s.at[idx_v.at[pl.ds(j * ROWS_PER_CHUNK, ROWS_PER_CHUNK)]],
                        bufs[b], sems[b]
                    ).wait()
                    reduce_chunk(bufs[b], obs[p], b * GN)

                    @pl.when(j + 2 < CHUNKS)
                    def _():
                        pltpu.async_copy(
                            x_s.at[idx_v.at[pl.ds((j + 2) * ROWS_PER_CHUNK,
                                                  ROWS_PER_CHUNK)]],
                            bufs[b], sems[b])
                pltpu.async_copy(
                    obs[p],
                    out_hbm.at[pl.ds(w * NODES_PER_W + base, 8)],
                    osems[p],
                )
            return 0

        lax.fori_loop(0, CHUNKS // 4, outer, 0)
        for p in range(2):
            pltpu.make_async_copy(obs[p], out_hbm.at[pl.ds(0, 8)], osems[p]).wait()

    return k(xp, idx_blocks)


def _tc_dense(nm_p, x, rf_perm, rf, w_self, w_neigh, bias2d):
    BLK = 1000
    grid = (N // BLK,)

    def body(nm_ref, x_ref, rfp_ref, rf_ref, ws_ref, wn_ref, b_ref, o_ref):
        a = jnp.dot(rfp_ref[...], wn_ref[...].T,
                    preferred_element_type=jnp.float32) * (1.0 / DEG)
        b = jnp.dot(rf_ref[...], ws_ref[...].T,
                    preferred_element_type=jnp.float32)
        y = (jnp.dot(nm_ref[...], a, preferred_element_type=jnp.float32)
             + jnp.dot(x_ref[...], b, preferred_element_type=jnp.float32)
             + b_ref[...])
        o_ref[...] = jnp.where(y > 0, y, jnp.exp(jnp.minimum(y, 0.0)) - 1.0)

    return pl.pallas_call(
        body,
        grid=grid,
        in_specs=[
            pl.BlockSpec((BLK, F), lambda i: (i, 0)),
            pl.BlockSpec((BLK, F), lambda i: (i, 0)),
            pl.BlockSpec((F, K * H), lambda i: (0, 0)),
            pl.BlockSpec((F, K * H), lambda i: (0, 0)),
            pl.BlockSpec((OUT, K * H), lambda i: (0, 0)),
            pl.BlockSpec((OUT, K * H), lambda i: (0, 0)),
            pl.BlockSpec((1, OUT), lambda i: (0, 0)),
        ],
        out_specs=pl.BlockSpec((BLK, OUT), lambda i: (i, 0)),
        out_shape=jax.ShapeDtypeStruct((N, OUT), jnp.float32),
    )(nm_p, x, rf_perm, rf, w_self, w_neigh, bias2d)


def kernel(x, node_ids, neighbor_idx, R, W_self, W_neigh, bias):
    del node_ids

    neigh_sum_p = _sc_segment_sum(x, neighbor_idx)
    r_flat = jnp.transpose(R, (1, 0, 2)).reshape(F, K * H)
    return _tc_dense(neigh_sum_p, x, r_flat, r_flat, W_self, W_neigh,
                     bias.reshape(1, OUT))


# speedup vs baseline: 1.0424x; 1.0424x over previous
import functools

import jax
import jax.numpy as jnp
import numpy as np
from jax import lax
from jax.experimental import pallas as pl
from jax.experimental.pallas import tpu as pltpu
from jax.experimental.pallas import tpu_sc as plsc

N = 10000
DEG = 32
F = 128
K = 4
H = 32
OUT = 128
FW = F // 2

NC = 2
NS = 16
NW = NC * NS
NODES_PER_W = 320
NP = NW * NODES_PER_W
GN = 2
ROWS_PER_CHUNK = GN * DEG
CHUNKS = NODES_PER_W // GN

_PERM = np.array(
    [32 * (p // 32) + (2 * (p % 32) if p % 32 < 16 else 2 * (p % 32 - 16) + 1)
     for p in range(F)], dtype=np.int32)


def _sc_segment_sum(xp, idx_blocks):
    mesh = plsc.VectorSubcoreMesh(
        core_axis_name="c", subcore_axis_name="s", num_cores=NC, num_subcores=NS
    )

    @functools.partial(
        pl.kernel,
        out_type=jax.ShapeDtypeStruct((NP, F), jnp.float32),
        mesh=mesh,
        scratch_types=[
            pltpu.VMEM_SHARED((N, F), jnp.float32),
            pltpu.VMEM((NODES_PER_W * DEG,), jnp.int32),
            pltpu.VMEM((ROWS_PER_CHUNK, F), jnp.float32),
            pltpu.VMEM((ROWS_PER_CHUNK, F), jnp.float32),
            pltpu.VMEM((ROWS_PER_CHUNK, F), jnp.float32),
            pltpu.VMEM((ROWS_PER_CHUNK, F), jnp.float32),
            pltpu.VMEM((8, F), jnp.float32),
            pltpu.VMEM((8, F), jnp.float32),
            pltpu.SemaphoreType.DMA,
            pltpu.SemaphoreType.DMA,
            pltpu.SemaphoreType.DMA,
            pltpu.SemaphoreType.DMA,
            pltpu.SemaphoreType.DMA,
            pltpu.SemaphoreType.DMA,
        ],
    )
    def k(x_hbm, idx_hbm, out_hbm, x_s, idx_v, buf0, buf1, buf2, buf3,
          ob0, ob1, sem0, sem1, sem2, sem3, osem0, osem1):
        s = lax.axis_index("s")
        w = s * NC + lax.axis_index("c")
        rows = 624
        pltpu.sync_copy(
            x_hbm.at[pl.ds(s * rows, rows)], x_s.at[pl.ds(s * rows, rows)]
        )

        @pl.when(s == 0)
        def _():
            pltpu.sync_copy(
                x_hbm.at[pl.ds(NS * rows, N - NS * rows)],
                x_s.at[pl.ds(NS * rows, N - NS * rows)],
            )
        IPW = NODES_PER_W * DEG

        @pl.when(w < NW - 1)
        def _():
            pltpu.sync_copy(idx_hbm.at[pl.ds(w * IPW, IPW)], idx_v)

        @pl.when(w == NW - 1)
        def _():
            real = N * DEG - (NW - 1) * IPW
            pltpu.sync_copy(
                idx_hbm.at[pl.ds((NW - 1) * IPW, real)], idx_v.at[pl.ds(0, real)]
            )
            zeros16 = jnp.zeros((16,), jnp.int32)

            def zfill(i, _):
                idx_v[pl.ds(real + i * 16, 16)] = zeros16
                return 0
            lax.fori_loop(0, (IPW - real) // 16, zfill, 0)
        plsc.subcore_barrier()
        bufs = (buf0, buf1, buf2, buf3)
        sems = (sem0, sem1, sem2, sem3)
        obs = (ob0, ob1)
        osems = (osem0, osem1)

        for b in range(4):
            pltpu.async_copy(
                x_s.at[idx_v.at[pl.ds(b * ROWS_PER_CHUNK, ROWS_PER_CHUNK)]],
                bufs[b], sems[b])

        def reduce_chunk(buf, ob, node0):
            for g in range(GN):
                def red(d, accs):
                    return tuple(
                        accs[cb] + buf[g * DEG + d, pl.ds(cb * 16, 16)]
                        for cb in range(8)
                    )
                accs = lax.fori_loop(
                    0, DEG, red,
                    tuple(jnp.zeros((16,), jnp.float32) for _ in range(8)),
                )
                for cb in range(8):
                    ob[node0 + g, pl.ds(cb * 16, 16)] = accs[cb]

        def outer(t2, _):
            for p in range(2):
                base = t2 * 16 + p * 8

                @pl.when(t2 > 0)
                def _():
                    pltpu.make_async_copy(
                        obs[p], out_hbm.at[pl.ds(0, 8)], osems[p]
                    ).wait()
                for b in range(4):
                    j = t2 * 8 + p * 4 + b
                    pltpu.make_async_copy(
                        x_s.at[idx_v.at[pl.ds(j * ROWS_PER_CHUNK, ROWS_PER_CHUNK)]],
                        bufs[b], sems[b]
                    ).wait()
                    reduce_chunk(bufs[b], obs[p], b * GN)

                    @pl.when(j + 4 < CHUNKS)
                    def _():
                        pltpu.async_copy(
                            x_s.at[idx_v.at[pl.ds((j + 4) * ROWS_PER_CHUNK,
                                                  ROWS_PER_CHUNK)]],
                            bufs[b], sems[b])
                pltpu.async_copy(
                    obs[p],
                    out_hbm.at[pl.ds(w * NODES_PER_W + base, 8)],
                    osems[p],
                )
            return 0

        lax.fori_loop(0, CHUNKS // 8, outer, 0)
        for p in range(2):
            pltpu.make_async_copy(obs[p], out_hbm.at[pl.ds(0, 8)], osems[p]).wait()

    return k(xp, idx_blocks)


def _tc_dense(nm_p, x, rf_perm, rf, w_self, w_neigh, bias2d):
    BLK = 1000
    grid = (N // BLK,)

    def body(nm_ref, x_ref, rfp_ref, rf_ref, ws_ref, wn_ref, b_ref, o_ref):
        a = jnp.dot(rfp_ref[...], wn_ref[...].T,
                    preferred_element_type=jnp.float32) * (1.0 / DEG)
        b = jnp.dot(rf_ref[...], ws_ref[...].T,
                    preferred_element_type=jnp.float32)
        y = (jnp.dot(nm_ref[...], a, preferred_element_type=jnp.float32)
             + jnp.dot(x_ref[...], b, preferred_element_type=jnp.float32)
             + b_ref[...])
        o_ref[...] = jnp.where(y > 0, y, jnp.exp(jnp.minimum(y, 0.0)) - 1.0)

    return pl.pallas_call(
        body,
        grid=grid,
        in_specs=[
            pl.BlockSpec((BLK, F), lambda i: (i, 0)),
            pl.BlockSpec((BLK, F), lambda i: (i, 0)),
            pl.BlockSpec((F, K * H), lambda i: (0, 0)),
            pl.BlockSpec((F, K * H), lambda i: (0, 0)),
            pl.BlockSpec((OUT, K * H), lambda i: (0, 0)),
            pl.BlockSpec((OUT, K * H), lambda i: (0, 0)),
            pl.BlockSpec((1, OUT), lambda i: (0, 0)),
        ],
        out_specs=pl.BlockSpec((BLK, OUT), lambda i: (i, 0)),
        out_shape=jax.ShapeDtypeStruct((N, OUT), jnp.float32),
    )(nm_p, x, rf_perm, rf, w_self, w_neigh, bias2d)


def kernel(x, node_ids, neighbor_idx, R, W_self, W_neigh, bias):
    del node_ids

    neigh_sum_p = _sc_segment_sum(x, neighbor_idx)
    r_flat = jnp.transpose(R, (1, 0, 2)).reshape(F, K * H)
    return _tc_dense(neigh_sum_p, x, r_flat, r_flat, W_self, W_neigh,
                     bias.reshape(1, OUT))


# speedup vs baseline: 1.0731x; 1.0294x over previous
import functools

import jax
import jax.numpy as jnp
import numpy as np
from jax import lax
from jax.experimental import pallas as pl
from jax.experimental.pallas import tpu as pltpu
from jax.experimental.pallas import tpu_sc as plsc

N = 10000
DEG = 32
F = 128
K = 4
H = 32
OUT = 128
FW = F // 2

NC = 2
NS = 16
NW = NC * NS
NODES_PER_W = 320
NP = NW * NODES_PER_W
GN = 2
ROWS_PER_CHUNK = GN * DEG
CHUNKS = NODES_PER_W // GN

_PERM = np.array(
    [32 * (p // 32) + (2 * (p % 32) if p % 32 < 16 else 2 * (p % 32 - 16) + 1)
     for p in range(F)], dtype=np.int32)


def _sc_segment_sum(xp, idx_blocks):
    mesh = plsc.VectorSubcoreMesh(
        core_axis_name="c", subcore_axis_name="s", num_cores=NC, num_subcores=NS
    )

    @functools.partial(
        pl.kernel,
        out_type=jax.ShapeDtypeStruct((NP, F), jnp.float32),
        mesh=mesh,
        scratch_types=[
            pltpu.VMEM_SHARED((N, F), jnp.float32),
            pltpu.VMEM((NODES_PER_W * DEG,), jnp.int32),
            pltpu.VMEM((ROWS_PER_CHUNK, F), jnp.float32),
            pltpu.VMEM((ROWS_PER_CHUNK, F), jnp.float32),
            pltpu.VMEM((ROWS_PER_CHUNK, F), jnp.float32),
            pltpu.VMEM((ROWS_PER_CHUNK, F), jnp.float32),
            pltpu.VMEM((8, F), jnp.float32),
            pltpu.VMEM((8, F), jnp.float32),
            pltpu.SemaphoreType.DMA,
            pltpu.SemaphoreType.DMA,
            pltpu.SemaphoreType.DMA,
            pltpu.SemaphoreType.DMA,
            pltpu.SemaphoreType.DMA,
            pltpu.SemaphoreType.DMA,
        ],
    )
    def k(x_hbm, idx_hbm, out_hbm, x_s, idx_v, buf0, buf1, buf2, buf3,
          ob0, ob1, sem0, sem1, sem2, sem3, osem0, osem1):
        s = lax.axis_index("s")
        w = s * NC + lax.axis_index("c")
        rows = 624
        pltpu.sync_copy(
            x_hbm.at[pl.ds(s * rows, rows)], x_s.at[pl.ds(s * rows, rows)]
        )

        @pl.when(s == 0)
        def _():
            pltpu.sync_copy(
                x_hbm.at[pl.ds(NS * rows, N - NS * rows)],
                x_s.at[pl.ds(NS * rows, N - NS * rows)],
            )
        IPW = NODES_PER_W * DEG

        @pl.when(w < NW - 1)
        def _():
            pltpu.sync_copy(idx_hbm.at[pl.ds(w * IPW, IPW)], idx_v)

        @pl.when(w == NW - 1)
        def _():
            real = N * DEG - (NW - 1) * IPW
            pltpu.sync_copy(
                idx_hbm.at[pl.ds((NW - 1) * IPW, real)], idx_v.at[pl.ds(0, real)]
            )
            zeros16 = jnp.zeros((16,), jnp.int32)

            def zfill(i, _):
                idx_v[pl.ds(real + i * 16, 16)] = zeros16
                return 0
            lax.fori_loop(0, (IPW - real) // 16, zfill, 0)
        plsc.subcore_barrier()
        bufs = (buf0, buf1, buf2, buf3)
        sems = (sem0, sem1, sem2, sem3)
        obs = (ob0, ob1)
        osems = (osem0, osem1)

        for b in range(4):
            pltpu.async_copy(
                x_s.at[idx_v.at[pl.ds(b * ROWS_PER_CHUNK, ROWS_PER_CHUNK)]],
                bufs[b], sems[b])

        def reduce_chunk(buf, ob, node0):
            for g in range(GN):
                def red(d, accs):
                    return tuple(
                        accs[cb] + buf[g * DEG + d, pl.ds(cb * 16, 16)]
                        for cb in range(8)
                    )
                accs = lax.fori_loop(
                    0, DEG, red,
                    tuple(jnp.zeros((16,), jnp.float32) for _ in range(8)),
                )
                for cb in range(8):
                    ob[node0 + g, pl.ds(cb * 16, 16)] = accs[cb]

        def outer(t2, _):
            for p in range(2):
                base = t2 * 16 + p * 8

                @pl.when(t2 > 0)
                def _():
                    pltpu.make_async_copy(
                        obs[p], out_hbm.at[pl.ds(0, 8)], osems[p]
                    ).wait()
                for b in range(4):
                    j = t2 * 8 + p * 4 + b
                    pltpu.make_async_copy(
                        x_s.at[idx_v.at[pl.ds(j * ROWS_PER_CHUNK, ROWS_PER_CHUNK)]],
                        bufs[b], sems[b]
                    ).wait()
                    reduce_chunk(bufs[b], obs[p], b * GN)

                    @pl.when(j + 4 < CHUNKS)
                    def _():
                        pltpu.async_copy(
                            x_s.at[idx_v.at[pl.ds((j + 4) * ROWS_PER_CHUNK,
                                                  ROWS_PER_CHUNK)]],
                            bufs[b], sems[b])
                pltpu.async_copy(
                    obs[p],
                    out_hbm.at[pl.ds(w * NODES_PER_W + base, 8)],
                    osems[p],
                )
            return 0

        lax.fori_loop(0, CHUNKS // 8, outer, 0)
        for p in range(2):
            pltpu.make_async_copy(obs[p], out_hbm.at[pl.ds(0, 8)], osems[p]).wait()

    return k(xp, idx_blocks)


def _tc_dense(nm_p, x, rf_perm, rf, w_self, w_neigh, bias2d):
    BLK = 2000
    grid = (N // BLK,)

    def body(nm_ref, x_ref, rfp_ref, rf_ref, ws_ref, wn_ref, b_ref, o_ref):
        a = jnp.dot(rfp_ref[...], wn_ref[...].T,
                    preferred_element_type=jnp.float32) * (1.0 / DEG)
        b = jnp.dot(rf_ref[...], ws_ref[...].T,
                    preferred_element_type=jnp.float32)
        y = (jnp.dot(nm_ref[...], a, preferred_element_type=jnp.float32)
             + jnp.dot(x_ref[...], b, preferred_element_type=jnp.float32)
             + b_ref[...])
        o_ref[...] = jnp.where(y > 0, y, jnp.exp(jnp.minimum(y, 0.0)) - 1.0)

    return pl.pallas_call(
        body,
        grid=grid,
        in_specs=[
            pl.BlockSpec((BLK, F), lambda i: (i, 0)),
            pl.BlockSpec((BLK, F), lambda i: (i, 0)),
            pl.BlockSpec((F, K * H), lambda i: (0, 0)),
            pl.BlockSpec((F, K * H), lambda i: (0, 0)),
            pl.BlockSpec((OUT, K * H), lambda i: (0, 0)),
            pl.BlockSpec((OUT, K * H), lambda i: (0, 0)),
            pl.BlockSpec((1, OUT), lambda i: (0, 0)),
        ],
        out_specs=pl.BlockSpec((BLK, OUT), lambda i: (i, 0)),
        out_shape=jax.ShapeDtypeStruct((N, OUT), jnp.float32),
    )(nm_p, x, rf_perm, rf, w_self, w_neigh, bias2d)


def kernel(x, node_ids, neighbor_idx, R, W_self, W_neigh, bias):
    del node_ids

    neigh_sum_p = _sc_segment_sum(x, neighbor_idx)
    r_flat = jnp.transpose(R, (1, 0, 2)).reshape(F, K * H)
    return _tc_dense(neigh_sum_p, x, r_flat, r_flat, W_self, W_neigh,
                     bias.reshape(1, OUT))


# speedup vs baseline: 1.0737x; 1.0005x over previous
import functools

import jax
import jax.numpy as jnp
from jax import lax
from jax.experimental import pallas as pl
from jax.experimental.pallas import tpu as pltpu
from jax.experimental.pallas import tpu_sc as plsc

N = 10000
DEG = 32
F = 128
K = 4
H = 32
OUT = 128

NC = 2
NS = 16
NW = NC * NS
NODES_PER_W = 320
NP = NW * NODES_PER_W
GN = 2
ROWS_PER_CHUNK = GN * DEG
CHUNKS = NODES_PER_W // GN


def _sc_segment_sum(xp, idx):
    mesh = plsc.VectorSubcoreMesh(
        core_axis_name="c", subcore_axis_name="s", num_cores=NC, num_subcores=NS
    )

    @functools.partial(
        pl.kernel,
        out_type=jax.ShapeDtypeStruct((NP, F), jnp.float32),
        mesh=mesh,
        scratch_types=[
            pltpu.VMEM_SHARED((N, F), jnp.float32),
            pltpu.VMEM((NODES_PER_W * DEG,), jnp.int32),
            pltpu.VMEM((ROWS_PER_CHUNK, F), jnp.float32),
            pltpu.VMEM((ROWS_PER_CHUNK, F), jnp.float32),
            pltpu.VMEM((ROWS_PER_CHUNK, F), jnp.float32),
            pltpu.VMEM((ROWS_PER_CHUNK, F), jnp.float32),
            pltpu.VMEM((8, F), jnp.float32),
            pltpu.VMEM((8, F), jnp.float32),
            pltpu.SemaphoreType.DMA,
            pltpu.SemaphoreType.DMA,
            pltpu.SemaphoreType.DMA,
            pltpu.SemaphoreType.DMA,
            pltpu.SemaphoreType.DMA,
            pltpu.SemaphoreType.DMA,
        ],
    )
    def k(x_hbm, idx_hbm, out_hbm, x_s, idx_v, buf0, buf1, buf2, buf3,
          ob0, ob1, sem0, sem1, sem2, sem3, osem0, osem1):
        s = lax.axis_index("s")
        w = s * NC + lax.axis_index("c")
        rows = 624
        pltpu.sync_copy(
            x_hbm.at[pl.ds(s * rows, rows)], x_s.at[pl.ds(s * rows, rows)]
        )

        @pl.when(s == 0)
        def _():
            pltpu.sync_copy(
                x_hbm.at[pl.ds(NS * rows, N - NS * rows)],
                x_s.at[pl.ds(NS * rows, N - NS * rows)],
            )
        IPW = NODES_PER_W * DEG

        @pl.when(w < NW - 1)
        def _():
            pltpu.sync_copy(idx_hbm.at[pl.ds(w * IPW, IPW)], idx_v)

        @pl.when(w == NW - 1)
        def _():
            real = N * DEG - (NW - 1) * IPW
            pltpu.sync_copy(
                idx_hbm.at[pl.ds((NW - 1) * IPW, real)], idx_v.at[pl.ds(0, real)]
            )
            zeros16 = jnp.zeros((16,), jnp.int32)

            def zfill(i, _):
                idx_v[pl.ds(real + i * 16, 16)] = zeros16
                return 0
            lax.fori_loop(0, (IPW - real) // 16, zfill, 0)
        plsc.subcore_barrier()
        bufs = (buf0, buf1, buf2, buf3)
        sems = (sem0, sem1, sem2, sem3)
        obs = (ob0, ob1)
        osems = (osem0, osem1)

        for b in range(4):
            pltpu.async_copy(
                x_s.at[idx_v.at[pl.ds(b * ROWS_PER_CHUNK, ROWS_PER_CHUNK)]],
                bufs[b], sems[b])

        def reduce_chunk(buf, ob, node0):
            for g in range(GN):
                def red(d, accs):
                    return tuple(
                        accs[cb] + buf[g * DEG + d, pl.ds(cb * 16, 16)]
                        for cb in range(8)
                    )
                accs = lax.fori_loop(
                    0, DEG, red,
                    tuple(jnp.zeros((16,), jnp.float32) for _ in range(8)),
                )
                for cb in range(8):
                    ob[node0 + g, pl.ds(cb * 16, 16)] = accs[cb]

        def outer(t2, _):
            for p in range(2):
                base = t2 * 16 + p * 8

                @pl.when(t2 > 0)
                def _():
                    pltpu.make_async_copy(
                        obs[p], out_hbm.at[pl.ds(0, 8)], osems[p]
                    ).wait()
                for b in range(4):
                    j = t2 * 8 + p * 4 + b
                    pltpu.make_async_copy(
                        x_s.at[idx_v.at[pl.ds(j * ROWS_PER_CHUNK, ROWS_PER_CHUNK)]],
                        bufs[b], sems[b]
                    ).wait()
                    reduce_chunk(bufs[b], obs[p], b * GN)

                    @pl.when(j + 4 < CHUNKS)
                    def _():
                        pltpu.async_copy(
                            x_s.at[idx_v.at[pl.ds((j + 4) * ROWS_PER_CHUNK,
                                                  ROWS_PER_CHUNK)]],
                            bufs[b], sems[b])
                pltpu.async_copy(
                    obs[p],
                    out_hbm.at[pl.ds(w * NODES_PER_W + base, 8)],
                    osems[p],
                )
            return 0

        lax.fori_loop(0, CHUNKS // 8, outer, 0)
        for p in range(2):
            pltpu.make_async_copy(obs[p], out_hbm.at[pl.ds(0, 8)], osems[p]).wait()

    return k(xp, idx)


def _tc_dense(nm_p, x, rf_a, rf, w_self, w_neigh, bias2d):
    BLK = 2000
    grid = (N // BLK,)

    def body(nm_ref, x_ref, rfp_ref, rf_ref, ws_ref, wn_ref, b_ref, o_ref):
        a = jnp.dot(rfp_ref[...], wn_ref[...].T,
                    preferred_element_type=jnp.float32) * (1.0 / DEG)
        b = jnp.dot(rf_ref[...], ws_ref[...].T,
                    preferred_element_type=jnp.float32)
        y = (jnp.dot(nm_ref[...], a, preferred_element_type=jnp.float32)
             + jnp.dot(x_ref[...], b, preferred_element_type=jnp.float32)
             + b_ref[...])
        o_ref[...] = jnp.where(y > 0, y, jnp.exp(jnp.minimum(y, 0.0)) - 1.0)

    return pl.pallas_call(
        body,
        grid=grid,
        in_specs=[
            pl.BlockSpec((BLK, F), lambda i: (i, 0)),
            pl.BlockSpec((BLK, F), lambda i: (i, 0)),
            pl.BlockSpec((F, K * H), lambda i: (0, 0)),
            pl.BlockSpec((F, K * H), lambda i: (0, 0)),
            pl.BlockSpec((OUT, K * H), lambda i: (0, 0)),
            pl.BlockSpec((OUT, K * H), lambda i: (0, 0)),
            pl.BlockSpec((1, OUT), lambda i: (0, 0)),
        ],
        out_specs=pl.BlockSpec((BLK, OUT), lambda i: (i, 0)),
        out_shape=jax.ShapeDtypeStruct((N, OUT), jnp.float32),
    )(nm_p, x, rf_a, rf, w_self, w_neigh, bias2d)


def kernel(x, node_ids, neighbor_idx, R, W_self, W_neigh, bias):
    del node_ids

    neigh_sum_p = _sc_segment_sum(x, neighbor_idx)
    r_flat = jnp.transpose(R, (1, 0, 2)).reshape(F, K * H)
    return _tc_dense(neigh_sum_p, x, r_flat, r_flat, W_self, W_neigh,
                     bias.reshape(1, OUT))
